# baseline (speedup 1.0000x reference)
import jax
import jax.numpy as jnp
from jax import lax
from jax.experimental import pallas as pl
from jax.experimental.pallas import tpu as pltpu
from jax.experimental.pallas import tpu_sc as plsc

NC = 2
NS = 16
NW = NC * NS
LANES = 16
CW = 16


def _prep_idx(src2, dst2, et2, n_nodes):
    eb, lw = src2.shape

    def body(s_ref, d_ref, t_ref, g_ref, x_ref):
        t = t_ref[...]
        g_ref[...] = t * n_nodes + s_ref[...]
        x_ref[...] = t * n_nodes + d_ref[...]

    return pl.pallas_call(
        body,
        out_shape=[jax.ShapeDtypeStruct((eb, lw), jnp.int32)] * 2,
    )(src2, dst2, et2)


def _count_sc(sidx3, zrows, ones_rows, rn, e):
    kc = sidx3.shape[2]
    nchunks = sidx3.shape[1]
    gf = 8
    tpr = rn // NS
    pch = 1000
    mesh = plsc.VectorSubcoreMesh(core_axis_name="c", subcore_axis_name="s")
    assert nchunks % gf == 0 and tpr % pch == 0

    def body(sidx_h, z_h, ones_h, out_h, idx_v, ones_v, w_v, acc_sh, sem):
        c = lax.axis_index("c")
        s = lax.axis_index("s")

        @pl.when(c == 0)
        def _():
            pltpu.sync_copy(ones_h, ones_v)
            pltpu.sync_copy(z_h, acc_sh.at[pl.ds(s * tpr, tpr)])
            pltpu.sync_copy(sidx_h.at[s], idx_v)
            plsc.subcore_barrier()

            def grp(g, _):
                cps = [
                    pltpu.async_copy(
                        ones_v, acc_sh.at[idx_v.at[g * gf + b]], sem,
                        add=True)
                    for b in range(gf)
                ]
                for cp in cps:
                    cp.wait()
                return 0

            lax.fori_loop(0, nchunks // gf, grp, 0)
            plsc.subcore_barrier()

            def piece(p, _):
                off = s * tpr + p * pch
                pltpu.sync_copy(acc_sh.at[pl.ds(off, pch)], w_v)

                def rowfn(i, _):
                    v = w_v[i, pl.ds(0, CW)]
                    w_v[i, pl.ds(0, CW)] = 1.0 / jnp.maximum(v, 1.0)
                    return 0

                lax.fori_loop(0, pch, rowfn, 0, unroll=4)
                pltpu.sync_copy(w_v, out_h.at[pl.ds(off, pch)])
                return 0

            lax.fori_loop(0, tpr // pch, piece, 0)

    k = pl.kernel(
        body,
        out_type=jax.ShapeDtypeStruct((rn, CW), jnp.float32),
        mesh=mesh,
        scratch_types=[
            pltpu.VMEM((nchunks, kc), jnp.int32),
            pltpu.VMEM((kc, CW), jnp.float32),
            pltpu.VMEM((pch, CW), jnp.float32),
            pltpu.VMEM_SHARED((rn, CW), jnp.float32),
            pltpu.SemaphoreType.DMA,
        ],
        compiler_params=pltpu.CompilerParams(use_tc_tiling_on_sc=False),
    )
    return k(sidx3, zrows, ones_rows)


def _y_tc(h, wl):
    n, d = h.shape
    r = wl.shape[0]
    bn = 1000

    def body(h_ref, w_ref, y_ref):
        y_ref[0] = jnp.dot(h_ref[...], w_ref[0],
                           preferred_element_type=jnp.float32)

    return pl.pallas_call(
        body,
        grid=(r, n // bn),
        in_specs=[
            pl.BlockSpec((bn, d), lambda ri, nb: (nb, 0)),
            pl.BlockSpec((1, d, d), lambda ri, nb: (ri, 0, 0)),
        ],
        out_specs=pl.BlockSpec((1, bn, d), lambda ri, nb: (ri, nb, 0)),
        out_shape=jax.ShapeDtypeStruct((r, n, d), jnp.float32),
    )(h, wl)


def _edge_sc(ytab, gidx3, sidx3, dst3, invc16, zrows, n, d, e):
    kk = gidx3.shape[2]
    nchunks = gidx3.shape[1]
    ngrp = 5
    gch = nchunks // ngrp
    nio = 10
    tpr = n // nio
    mesh = plsc.VectorSubcoreMesh(core_axis_name="c", subcore_axis_name="s")
    assert gch * ngrp == nchunks and gch % 2 == 1 and gch >= 3

    def body(ytab_h, gidx_h, sidx_h, dst_h, invc_h, z_h, out_h,
             gidxA_v, sidxA_v, dstA_v, gidxB_v, sidxB_v, dstB_v,
             rows0_v, rows1_v, scl0_v, scl1_v,
             acc_sh, semr0, semr1, sems0, sems1, semi):
        c = lax.axis_index("c")
        s = lax.axis_index("s")
        wid = c * NS + s

        @pl.when(s < nio)
        def _():
            pltpu.sync_copy(z_h, acc_sh.at[pl.ds(s * tpr, tpr)])

        plsc.subcore_barrier()

        bufs = ((rows0_v, scl0_v, semr0, sems0),
                (rows1_v, scl1_v, semr1, sems1))
        idx_sets = ((gidxA_v, sidxA_v, dstA_v),
                    (gidxB_v, sidxB_v, dstB_v))

        def refill(grp, iset, fire):
            off = grp * gch
            for href, vref in zip((gidx_h, sidx_h, dst_h), idx_sets[iset]):
                cp = pltpu.make_async_copy(
                    href.at[wid, pl.ds(off, gch)], vref, semi)
                if fire:
                    cp.start()
                else:
                    cp.wait()

        def start_g(i, b, iset):
            gidx_v, sidx_v, _ = idx_sets[iset]
            rows_b, scl_b, semr, sems = bufs[b]
            pltpu.make_async_copy(
                ytab_h.at[gidx_v.at[i]], rows_b, semr).start()
            pltpu.make_async_copy(
                invc_h.at[sidx_v.at[i]], scl_b, sems).start()

        def proc(i, b, iset):
            gidx_v, sidx_v, dst_v = idx_sets[iset]
            rows_b, scl_b, semr, sems = bufs[b]
            pltpu.make_async_copy(
                ytab_h.at[gidx_v.at[i]], rows_b, semr).wait()
            pltpu.make_async_copy(
                invc_h.at[sidx_v.at[i]], scl_b, sems).wait()

            def rowfn(row, _):
                svec = scl_b[row, pl.ds(0, LANES)]
                for i8 in range(d // LANES):
                    sl = pl.ds(i8 * LANES, LANES)
                    rows_b[row, sl] = rows_b[row, sl] * svec
                return 0

            lax.fori_loop(0, kk, rowfn, 0, unroll=8)
            pltpu.sync_copy(rows_b, acc_sh.at[dst_v.at[i]], add=True)

        refill(0, 0, True)
        for grp in range(ngrp):
            iset = grp % 2
            refill(grp, iset, False)
            if grp + 1 < ngrp:
                refill(grp + 1, 1 - iset, True)
            start_g(0, 0, iset)

            def jbody(j, _, iset=iset):
                i0 = 2 * j
                start_g(i0 + 1, 1, iset)
                proc(i0, 0, iset)
                start_g(i0 + 2, 0, iset)
                proc(i0 + 1, 1, iset)
                return 0

            lax.fori_loop(0, (gch - 1) // 2, jbody, 0)
            proc(gch - 1, 0, iset)

        plsc.subcore_barrier()

        @pl.when(s < nio)
        def _():
            pltpu.sync_copy(acc_sh.at[pl.ds(s * tpr, tpr)],
                            out_h.at[c, pl.ds(s * tpr, tpr)])

    k = pl.kernel(
        body,
        out_type=jax.ShapeDtypeStruct((NC, n, d), jnp.float32),
        mesh=mesh,
        scratch_types=[
            pltpu.VMEM((nchunks // 5, kk), jnp.int32),
            pltpu.VMEM((nchunks // 5, kk), jnp.int32),
            pltpu.VMEM((nchunks // 5, kk), jnp.int32),
            pltpu.VMEM((nchunks // 5, kk), jnp.int32),
            pltpu.VMEM((nchunks // 5, kk), jnp.int32),
            pltpu.VMEM((nchunks // 5, kk), jnp.int32),
            pltpu.VMEM((kk, d), jnp.float32),
            pltpu.VMEM((kk, d), jnp.float32),
            pltpu.VMEM((kk, CW), jnp.float32),
            pltpu.VMEM((kk, CW), jnp.float32),
            pltpu.VMEM_SHARED((n, d), jnp.float32),
            pltpu.SemaphoreType.DMA,
            pltpu.SemaphoreType.DMA,
            pltpu.SemaphoreType.DMA,
            pltpu.SemaphoreType.DMA,
            pltpu.SemaphoreType.DMA,
        ],
        compiler_params=pltpu.CompilerParams(use_tc_tiling_on_sc=False),
    )
    return k(ytab, gidx3, sidx3, dst3, invc16, zrows)


def _combine_y_tc(parts, h, rootl, bias2, wnext):
    n, d = h.shape
    r = wnext.shape[0]
    bn = 1000

    def body(p_ref, h_ref, r_ref, b_ref, w_ref, hn_ref, y_ref):
        hb = (p_ref[0] + p_ref[1]
              + jnp.dot(h_ref[...], r_ref[...],
                        preferred_element_type=jnp.float32)
              + b_ref[...])
        hn_ref[...] = hb
        for ri in range(r):
            y_ref[ri] = jnp.dot(hb, w_ref[ri],
                                preferred_element_type=jnp.float32)

    return pl.pallas_call(
        body,
        grid=(n // bn,),
        in_specs=[
            pl.BlockSpec((NC, bn, d), lambda nb: (0, nb, 0)),
            pl.BlockSpec((bn, d), lambda nb: (nb, 0)),
            pl.BlockSpec((d, d), lambda nb: (0, 0)),
            pl.BlockSpec((1, d), lambda nb: (0, 0)),
            pl.BlockSpec((r, d, d), lambda nb: (0, 0, 0)),
        ],
        out_specs=[
            pl.BlockSpec((bn, d), lambda nb: (nb, 0)),
            pl.BlockSpec((r, bn, d), lambda nb: (0, nb, 0)),
        ],
        out_shape=[
            jax.ShapeDtypeStruct((n, d), jnp.float32),
            jax.ShapeDtypeStruct((r, n, d), jnp.float32),
        ],
    )(parts, h, rootl, bias2, wnext)


def _combine_tc(parts, h, rootl, bias2):
    n, d = h.shape
    bn = 1000

    def body(p_ref, h_ref, r_ref, b_ref, o_ref):
        hb = h_ref[...]
        o_ref[...] = (p_ref[0] + p_ref[1]
                      + jnp.dot(hb, r_ref[...],
                                preferred_element_type=jnp.float32)
                      + b_ref[...])

    return pl.pallas_call(
        body,
        grid=(n // bn,),
        in_specs=[
            pl.BlockSpec((NC, bn, d), lambda nb: (0, nb, 0)),
            pl.BlockSpec((bn, d), lambda nb: (nb, 0)),
            pl.BlockSpec((d, d), lambda nb: (0, 0)),
            pl.BlockSpec((1, d), lambda nb: (0, 0)),
        ],
        out_specs=pl.BlockSpec((bn, d), lambda nb: (nb, 0)),
        out_shape=jax.ShapeDtypeStruct((n, d), jnp.float32),
    )(parts, h, rootl, bias2)


@jax.jit
def kernel(x, edge_index, edge_type, W, root, bias):
    n, d = x.shape
    e = edge_index.shape[1]
    num_layers, r = W.shape[0], W.shape[1]
    rn = r * n

    src = edge_index[0]
    dst = edge_index[1]
    eb = e // 128
    gidx2, sidx2 = _prep_idx(src.reshape(eb, 128), dst.reshape(eb, 128),
                             edge_type.reshape(eb, 128), n)
    gidx = gidx2.reshape(e)
    sidx = sidx2.reshape(e)

    zrows_rn = jnp.zeros((rn // NS, CW), jnp.float32)
    ones_rows = jnp.ones((125, CW), jnp.float32)
    sidx3c = sidx.reshape(NS, e // NS // 125, 125)
    invc16 = _count_sc(sidx3c, zrows_rn, ones_rows, rn, e)

    kk = 80
    nchunks = e // NW // kk
    gidx3 = gidx.reshape(NW, nchunks, kk)
    sidx3 = sidx.reshape(NW, nchunks, kk)
    dst3 = dst.reshape(NW, nchunks, kk)

    zrows = jnp.zeros((n // 10, d), jnp.float32)
    bias2 = bias.reshape(num_layers, 1, d)

    h = x
    y = _y_tc(h, W[0]).reshape(rn, d)
    for l in range(num_layers - 1):
        parts = _edge_sc(y, gidx3, sidx3, dst3, invc16, zrows, n, d, e)
        h, y = _combine_y_tc(parts, h, root[l], bias2[l], W[l + 1])
        y = y.reshape(rn, d)
    parts = _edge_sc(y, gidx3, sidx3, dst3, invc16, zrows, n, d, e)
    return _combine_tc(parts, h, root[num_layers - 1],
                       bias2[num_layers - 1])


# speedup vs baseline: 1.1205x; 1.1205x over previous
import jax
import jax.numpy as jnp
from jax import lax
from jax.experimental import pallas as pl
from jax.experimental.pallas import tpu as pltpu
from jax.experimental.pallas import tpu_sc as plsc

NC = 2
NS = 16
NW = NC * NS
LANES = 16
CW = 16


def _prep_idx(src2, dst2, et2, n_nodes):
    eb, lw = src2.shape

    def body(s_ref, d_ref, t_ref, g_ref, x_ref):
        t = t_ref[...]
        g_ref[...] = t * n_nodes + s_ref[...]
        x_ref[...] = t * n_nodes + d_ref[...]

    return pl.pallas_call(
        body,
        out_shape=[jax.ShapeDtypeStruct((eb, lw), jnp.int32)] * 2,
    )(src2, dst2, et2)


def _count_sc(sidx3, zrows, ones_rows, rn, e):
    kc = sidx3.shape[2]
    nchunks = sidx3.shape[1]
    gf = 8
    tpr = rn // NS
    pch = 1000
    mesh = plsc.VectorSubcoreMesh(core_axis_name="c", subcore_axis_name="s")
    assert nchunks % gf == 0 and tpr % pch == 0

    def body(sidx_h, z_h, ones_h, out_h, idx_v, ones_v, w_v, acc_sh, sem):
        c = lax.axis_index("c")
        s = lax.axis_index("s")

        @pl.when(c == 0)
        def _():
            pltpu.sync_copy(ones_h, ones_v)
            pltpu.sync_copy(z_h, acc_sh.at[pl.ds(s * tpr, tpr)])
            pltpu.sync_copy(sidx_h.at[s], idx_v)
            plsc.subcore_barrier()

            def grp(g, _):
                cps = [
                    pltpu.async_copy(
                        ones_v, acc_sh.at[idx_v.at[g * gf + b]], sem,
                        add=True)
                    for b in range(gf)
                ]
                for cp in cps:
                    cp.wait()
                return 0

            lax.fori_loop(0, nchunks // gf, grp, 0)
            plsc.subcore_barrier()

            def piece(p, _):
                off = s * tpr + p * pch
                pltpu.sync_copy(acc_sh.at[pl.ds(off, pch)], w_v)

                def rowfn(i, _):
                    v = w_v[i, pl.ds(0, CW)]
                    w_v[i, pl.ds(0, CW)] = 1.0 / jnp.maximum(v, 1.0)
                    return 0

                lax.fori_loop(0, pch, rowfn, 0, unroll=4)
                pltpu.sync_copy(w_v, out_h.at[pl.ds(off, pch)])
                return 0

            lax.fori_loop(0, tpr // pch, piece, 0)

    k = pl.kernel(
        body,
        out_type=jax.ShapeDtypeStruct((rn, CW), jnp.float32),
        mesh=mesh,
        scratch_types=[
            pltpu.VMEM((nchunks, kc), jnp.int32),
            pltpu.VMEM((kc, CW), jnp.float32),
            pltpu.VMEM((pch, CW), jnp.float32),
            pltpu.VMEM_SHARED((rn, CW), jnp.float32),
            pltpu.SemaphoreType.DMA,
        ],
        compiler_params=pltpu.CompilerParams(use_tc_tiling_on_sc=False),
    )
    return k(sidx3, zrows, ones_rows)


def _y_tc(h, wl):
    n, d = h.shape
    r = wl.shape[0]
    bn = 1000

    def body(h_ref, w_ref, y_ref):
        y_ref[0] = jnp.dot(h_ref[...], w_ref[0],
                           preferred_element_type=jnp.float32)

    return pl.pallas_call(
        body,
        grid=(r, n // bn),
        in_specs=[
            pl.BlockSpec((bn, d), lambda ri, nb: (nb, 0)),
            pl.BlockSpec((1, d, d), lambda ri, nb: (ri, 0, 0)),
        ],
        out_specs=pl.BlockSpec((1, bn, d), lambda ri, nb: (ri, nb, 0)),
        out_shape=jax.ShapeDtypeStruct((r, n, d), jnp.float32),
    )(h, wl)


def _edge_sc(ytab, gidx3, sidx3, dst3, invc16, zrows, n, d, e):
    kk = gidx3.shape[2]
    nchunks = gidx3.shape[1]
    ngrp = 5
    gch = nchunks // ngrp
    nio = 10
    tpr = n // nio
    mesh = plsc.VectorSubcoreMesh(core_axis_name="c", subcore_axis_name="s")
    assert gch * ngrp == nchunks and gch % 2 == 1 and gch >= 3

    def body(ytab_h, gidx_h, sidx_h, dst_h, invc_h, z_h, out_h,
             gidxA_v, sidxA_v, dstA_v, gidxB_v, sidxB_v, dstB_v,
             rows0_v, rows1_v, scl0_v, scl1_v,
             acc_sh, semr0, semr1, sems0, sems1, semi):
        c = lax.axis_index("c")
        s = lax.axis_index("s")
        wid = c * NS + s

        @pl.when(s < nio)
        def _():
            pltpu.sync_copy(z_h, acc_sh.at[pl.ds(s * tpr, tpr)])

        plsc.subcore_barrier()

        bufs = ((rows0_v, scl0_v, semr0, sems0),
                (rows1_v, scl1_v, semr1, sems1))
        idx_sets = ((gidxA_v, sidxA_v, dstA_v),
                    (gidxB_v, sidxB_v, dstB_v))

        def refill(grp, iset, fire):
            off = grp * gch
            for href, vref in zip((gidx_h, sidx_h, dst_h), idx_sets[iset]):
                cp = pltpu.make_async_copy(
                    href.at[wid, pl.ds(off, gch)], vref, semi)
                if fire:
                    cp.start()
                else:
                    cp.wait()

        def start_g(i, b, iset):
            gidx_v, sidx_v, _ = idx_sets[iset]
            rows_b, scl_b, semr, sems = bufs[b]
            pltpu.make_async_copy(
                ytab_h.at[gidx_v.at[i]], rows_b, semr).start()
            pltpu.make_async_copy(
                invc_h.at[sidx_v.at[i]], scl_b, sems).start()

        def proc(i, b, iset):
            gidx_v, sidx_v, dst_v = idx_sets[iset]
            rows_b, scl_b, semr, sems = bufs[b]
            pltpu.make_async_copy(
                ytab_h.at[gidx_v.at[i]], rows_b, semr).wait()
            pltpu.make_async_copy(
                invc_h.at[sidx_v.at[i]], scl_b, sems).wait()

            def rowfn(row, _):
                svec = scl_b[row, pl.ds(0, LANES)]
                for i8 in range(d // LANES):
                    sl = pl.ds(i8 * LANES, LANES)
                    rows_b[row, sl] = rows_b[row, sl] * svec
                return 0

            lax.fori_loop(0, kk, rowfn, 0, unroll=4)
            pltpu.sync_copy(rows_b, acc_sh.at[dst_v.at[i]], add=True)

        refill(0, 0, True)
        for grp in range(ngrp):
            iset = grp % 2
            refill(grp, iset, False)
            if grp + 1 < ngrp:
                refill(grp + 1, 1 - iset, True)
            start_g(0, 0, iset)

            def jbody(j, _, iset=iset):
                i0 = 2 * j
                start_g(i0 + 1, 1, iset)
                proc(i0, 0, iset)
                start_g(i0 + 2, 0, iset)
                proc(i0 + 1, 1, iset)
                return 0

            lax.fori_loop(0, (gch - 1) // 2, jbody, 0)
            proc(gch - 1, 0, iset)

        plsc.subcore_barrier()

        @pl.when(s < nio)
        def _():
            pltpu.sync_copy(acc_sh.at[pl.ds(s * tpr, tpr)],
                            out_h.at[c, pl.ds(s * tpr, tpr)])

    k = pl.kernel(
        body,
        out_type=jax.ShapeDtypeStruct((NC, n, d), jnp.float32),
        mesh=mesh,
        scratch_types=[
            pltpu.VMEM((nchunks // 5, kk), jnp.int32),
            pltpu.VMEM((nchunks // 5, kk), jnp.int32),
            pltpu.VMEM((nchunks // 5, kk), jnp.int32),
            pltpu.VMEM((nchunks // 5, kk), jnp.int32),
            pltpu.VMEM((nchunks // 5, kk), jnp.int32),
            pltpu.VMEM((nchunks // 5, kk), jnp.int32),
            pltpu.VMEM((kk, d), jnp.float32),
            pltpu.VMEM((kk, d), jnp.float32),
            pltpu.VMEM((kk, CW), jnp.float32),
            pltpu.VMEM((kk, CW), jnp.float32),
            pltpu.VMEM_SHARED((n, d), jnp.float32),
            pltpu.SemaphoreType.DMA,
            pltpu.SemaphoreType.DMA,
            pltpu.SemaphoreType.DMA,
            pltpu.SemaphoreType.DMA,
            pltpu.SemaphoreType.DMA,
        ],
        compiler_params=pltpu.CompilerParams(use_tc_tiling_on_sc=False),
    )
    return k(ytab, gidx3, sidx3, dst3, invc16, zrows)


def _combine_y_tc(parts, h, rootl, bias2, wnext):
    n, d = h.shape
    r = wnext.shape[0]
    bn = 1000

    def body(p_ref, h_ref, r_ref, b_ref, w_ref, hn_ref, y_ref):
        hb = (p_ref[0] + p_ref[1]
              + jnp.dot(h_ref[...], r_ref[...],
                        preferred_element_type=jnp.float32)
              + b_ref[...])
        hn_ref[...] = hb
        for ri in range(r):
            y_ref[ri] = jnp.dot(hb, w_ref[ri],
                                preferred_element_type=jnp.float32)

    return pl.pallas_call(
        body,
        grid=(n // bn,),
        in_specs=[
            pl.BlockSpec((NC, bn, d), lambda nb: (0, nb, 0)),
            pl.BlockSpec((bn, d), lambda nb: (nb, 0)),
            pl.BlockSpec((d, d), lambda nb: (0, 0)),
            pl.BlockSpec((1, d), lambda nb: (0, 0)),
            pl.BlockSpec((r, d, d), lambda nb: (0, 0, 0)),
        ],
        out_specs=[
            pl.BlockSpec((bn, d), lambda nb: (nb, 0)),
            pl.BlockSpec((r, bn, d), lambda nb: (0, nb, 0)),
        ],
        out_shape=[
            jax.ShapeDtypeStruct((n, d), jnp.float32),
            jax.ShapeDtypeStruct((r, n, d), jnp.float32),
        ],
    )(parts, h, rootl, bias2, wnext)


def _combine_tc(parts, h, rootl, bias2):
    n, d = h.shape
    bn = 1000

    def body(p_ref, h_ref, r_ref, b_ref, o_ref):
        hb = h_ref[...]
        o_ref[...] = (p_ref[0] + p_ref[1]
                      + jnp.dot(hb, r_ref[...],
                                preferred_element_type=jnp.float32)
                      + b_ref[...])

    return pl.pallas_call(
        body,
        grid=(n // bn,),
        in_specs=[
            pl.BlockSpec((NC, bn, d), lambda nb: (0, nb, 0)),
            pl.BlockSpec((bn, d), lambda nb: (nb, 0)),
            pl.BlockSpec((d, d), lambda nb: (0, 0)),
            pl.BlockSpec((1, d), lambda nb: (0, 0)),
        ],
        out_specs=pl.BlockSpec((bn, d), lambda nb: (nb, 0)),
        out_shape=jax.ShapeDtypeStruct((n, d), jnp.float32),
    )(parts, h, rootl, bias2)


@jax.jit
def kernel(x, edge_index, edge_type, W, root, bias):
    n, d = x.shape
    e = edge_index.shape[1]
    num_layers, r = W.shape[0], W.shape[1]
    rn = r * n

    src = edge_index[0]
    dst = edge_index[1]
    eb = e // 128
    gidx2, sidx2 = _prep_idx(src.reshape(eb, 128), dst.reshape(eb, 128),
                             edge_type.reshape(eb, 128), n)
    gidx = gidx2.reshape(e)
    sidx = sidx2.reshape(e)

    zrows_rn = jnp.zeros((rn // NS, CW), jnp.float32)
    ones_rows = jnp.ones((125, CW), jnp.float32)
    sidx3c = sidx.reshape(NS, e // NS // 125, 125)
    invc16 = _count_sc(sidx3c, zrows_rn, ones_rows, rn, e)

    kk = 80
    nchunks = e // NW // kk
    gidx3 = gidx.reshape(NW, nchunks, kk)
    sidx3 = sidx.reshape(NW, nchunks, kk)
    dst3 = dst.reshape(NW, nchunks, kk)

    zrows = jnp.zeros((n // 10, d), jnp.float32)
    bias2 = bias.reshape(num_layers, 1, d)

    h = x
    y = _y_tc(h, W[0]).reshape(rn, d)
    for l in range(num_layers - 1):
        parts = _edge_sc(y, gidx3, sidx3, dst3, invc16, zrows, n, d, e)
        h, y = _combine_y_tc(parts, h, root[l], bias2[l], W[l + 1])
        y = y.reshape(rn, d)
    parts = _edge_sc(y, gidx3, sidx3, dst3, invc16, zrows, n, d, e)
    return _combine_tc(parts, h, root[num_layers - 1],
                       bias2[num_layers - 1])
